# Initial kernel scaffold; baseline (speedup 1.0000x reference)
import jax
import jax.numpy as jnp
from jax.experimental import pallas as pl


def kernel(pos, pos_encoding):
    raise NotImplementedError("write your pallas kernel here")


# speedup vs baseline: 12.7033x; 12.7033x over previous
import functools

import jax
import jax.numpy as jnp
from jax import lax
from jax.experimental import pallas as pl
from jax.experimental.pallas import tpu as pltpu
from jax.experimental.pallas import tpu_sc as plsc

_DIM = 64
_NC = 2
_NS = 16
_NW = _NC * _NS
_SUB = 128
_K = 4
_CHUNK = _SUB * _K


def _gather_sc(table, idx2d, n):
    per_w = n // _NW
    chunks = per_w // _CHUNK
    idx_rows_per_w = per_w // _SUB

    mesh = plsc.VectorSubcoreMesh(core_axis_name="c", subcore_axis_name="s")

    @functools.partial(
        pl.kernel,
        mesh=mesh,
        compiler_params=pltpu.CompilerParams(use_tc_tiling_on_sc=False),
        out_type=jax.ShapeDtypeStruct((n, _DIM), jnp.float32),
        scratch_types=[
            pltpu.VMEM((_K, _SUB), jnp.int32),
            pltpu.VMEM((_CHUNK, _DIM), jnp.float32),
            pltpu.SemaphoreType.DMA,
        ],
    )
    def k(table_hbm, idx_hbm, out_hbm, idx_v, rows_v, sem):
        wid = lax.axis_index("s") * _NC + lax.axis_index("c")
        row0 = wid * idx_rows_per_w

        def body(i, carry):
            idx_row = row0 + i * _K
            pltpu.sync_copy(idx_hbm.at[pl.ds(idx_row, _K)], idx_v)
            copies = [
                pltpu.async_copy(
                    table_hbm.at[idx_v.at[j]],
                    rows_v.at[pl.ds(j * _SUB, _SUB)],
                    sem,
                )
                for j in range(_K)
            ]
            for c in copies:
                c.wait()
            pltpu.sync_copy(rows_v, out_hbm.at[pl.ds(idx_row * _SUB, _CHUNK)])
            return carry

        lax.fori_loop(0, chunks, body, 0)

    return k(table, idx2d)


def kernel(pos, pos_encoding):
    b, l = pos.shape
    n = b * l
    assert n % (_NW * _CHUNK) == 0
    idx2d = pos.reshape(n // _SUB, _SUB)
    out = _gather_sc(pos_encoding, idx2d, n)
    return out.reshape(b, l, _DIM)


# speedup vs baseline: 13.5604x; 1.0675x over previous
import functools

import jax
import jax.numpy as jnp
from jax import lax
from jax.experimental import pallas as pl
from jax.experimental.pallas import tpu as pltpu
from jax.experimental.pallas import tpu_sc as plsc

_DIM = 64
_NC = 2
_NS = 16
_NW = _NC * _NS
_SUB = 128
_K = 4
_CHUNK = _SUB * _K


def _gather_sc(table, idx2d, n):
    per_w = n // _NW
    chunks = per_w // _CHUNK
    idx_rows_per_w = per_w // _SUB

    mesh = plsc.VectorSubcoreMesh(core_axis_name="c", subcore_axis_name="s")

    @functools.partial(
        pl.kernel,
        mesh=mesh,
        compiler_params=pltpu.CompilerParams(use_tc_tiling_on_sc=False),
        out_type=jax.ShapeDtypeStruct((n, _DIM), jnp.float32),
        scratch_types=[
            pltpu.VMEM((idx_rows_per_w, _SUB), jnp.int32),
            pltpu.VMEM((2, _CHUNK, _DIM), jnp.float32),
            pltpu.SemaphoreType.DMA,
            pltpu.SemaphoreType.DMA,
        ],
    )
    def k(table_hbm, idx_hbm, out_hbm, idx_v, rows_v, gsem, osem):
        wid = lax.axis_index("s") * _NC + lax.axis_index("c")
        row0 = wid * idx_rows_per_w
        pltpu.sync_copy(idx_hbm.at[pl.ds(row0, idx_rows_per_w)], idx_v)

        def body(g, carry):
            buf = rows_v.at[g % 2]
            out_off = (row0 + g * _K) * _SUB

            @pl.when(g >= 2)
            def _():
                pltpu.make_async_copy(
                    buf, out_hbm.at[pl.ds(out_off, _CHUNK)], osem
                ).wait()

            for j in range(_K):
                pltpu.async_copy(
                    table_hbm.at[idx_v.at[g * _K + j]],
                    buf.at[pl.ds(j * _SUB, _SUB)],
                    gsem,
                )
            pltpu.make_async_copy(
                table_hbm.at[idx_v.at[0]], buf, gsem
            ).wait()

            pltpu.async_copy(buf, out_hbm.at[pl.ds(out_off, _CHUNK)], osem)
            return carry

        lax.fori_loop(0, chunks, body, 0)

        for b in range(2):
            pltpu.make_async_copy(
                rows_v.at[b], out_hbm.at[pl.ds(row0 * _SUB, _CHUNK)], osem
            ).wait()

    return k(table, idx2d)


def kernel(pos, pos_encoding):
    b, l = pos.shape
    n = b * l
    assert n % (_NW * _CHUNK) == 0
    idx2d = pos.reshape(n // _SUB, _SUB)
    out = _gather_sc(pos_encoding, idx2d, n)
    return out.reshape(b, l, _DIM)
